# baseline (speedup 1.0000x reference)
import functools
import math

import jax
import jax.numpy as jnp
from jax import lax
from jax.experimental import pallas as pl
from jax.experimental.pallas import tpu as pltpu
from jax.experimental.pallas import tpu_sc as plsc

B, N = 8, 2048
DP, DM, K, H = 64, 128, 16, 4
HD = DM // H

QA = 256
QB = 128
RB = QB * K


CH_A = 8


def _stage_a_body(xyzq_ref, xyzt_ref, feat_ref, fc1_ref, fc1b_ref,
                  wq_ref, wk_ref, wv_ref, q_ref, tbl_ref, knn_ref, d_ref):
    b = pl.program_id(0)
    x = jnp.dot(feat_ref[0], fc1_ref[...]) + fc1b_ref[...]
    q_ref[0] = jnp.dot(x, wq_ref[...])
    xk = jnp.dot(x, wk_ref[...])
    xv = jnp.dot(x, wv_ref[...])

    xq = xyzq_ref[0]
    xt = xyzt_ref[0]
    tbl_ref[0] = jnp.concatenate(
        [xk, xv, xq, jnp.zeros((QA, DM - 3), jnp.float32)], axis=1)

    qn = jnp.sum(xq * xq, axis=1, keepdims=True)
    kn = jnp.sum(xt * xt, axis=0, keepdims=True)
    cross = lax.dot_general(xq, xt, (((1,), (0,)), ((), ())),
                            precision=lax.Precision.DEFAULT)
    d_ref[...] = qn + kn - 2.0 * cross

    def chunk(c, carry):
        d = d_ref[pl.ds(c * CH_A, CH_A), :]
        iota = lax.broadcasted_iota(jnp.int32, (CH_A, N), 1)
        cols = []
        for j in range(K):
            m = jnp.min(d, axis=1, keepdims=True)
            idx = jnp.min(jnp.where(d == m, iota, N), axis=1, keepdims=True)
            cols.append(idx)
            d = jnp.where(iota == idx, jnp.float32(jnp.inf), d)
        knn_ref[0, pl.ds(c * CH_A, CH_A), :] = (
            jnp.concatenate(cols, axis=1) + b * N)
        return carry

    lax.fori_loop(0, QA // CH_A, chunk, 0)


def _stage_a(xyz, xyzt, feats, p):
    grid = (B, N // QA)
    out = pl.pallas_call(
        _stage_a_body,
        grid=grid,
        in_specs=[
            pl.BlockSpec((1, QA, 3), lambda b, i: (b, i, 0)),
            pl.BlockSpec((1, 3, N), lambda b, i: (b, 0, 0)),
            pl.BlockSpec((1, QA, DP), lambda b, i: (b, i, 0)),
            pl.BlockSpec((DP, DM), lambda b, i: (0, 0)),
            pl.BlockSpec((1, DM), lambda b, i: (0, 0)),
            pl.BlockSpec((DM, DM), lambda b, i: (0, 0)),
            pl.BlockSpec((DM, DM), lambda b, i: (0, 0)),
            pl.BlockSpec((DM, DM), lambda b, i: (0, 0)),
        ],
        out_specs=[
            pl.BlockSpec((1, QA, DM), lambda b, i: (b, i, 0)),
            pl.BlockSpec((1, QA, 3 * DM), lambda b, i: (b, i, 0)),
            pl.BlockSpec((1, QA, K), lambda b, i: (b, i, 0)),
        ],
        out_shape=[
            jax.ShapeDtypeStruct((B, N, DM), jnp.float32),
            jax.ShapeDtypeStruct((B, N, 3 * DM), jnp.float32),
            jax.ShapeDtypeStruct((B, N, K), jnp.int32),
        ],
        scratch_shapes=[pltpu.VMEM((QA, N), jnp.float32)],
    )(xyz, xyzt, feats, p['fc1_w'], p['fc1_b'].reshape(1, DM),
      p['wq'], p['wk'], p['wv'])
    return out


def _sc_gather(tbl2, idx3, nc, nw):
    tot = B * N * K
    per_w = tot // nw
    ch = 128
    nch = per_w // ch

    mesh = plsc.VectorSubcoreMesh(core_axis_name="c", subcore_axis_name="s")

    @functools.partial(
        pl.kernel,
        mesh=mesh,
        out_type=jax.ShapeDtypeStruct((tot, 3 * DM), jnp.float32),
        scratch_types=[
            pltpu.VMEM((nch, ch), jnp.int32),
            pltpu.VMEM((ch, 3 * DM), jnp.float32),
            pltpu.VMEM((ch, 3 * DM), jnp.float32),
            pltpu.SemaphoreType.DMA,
            pltpu.SemaphoreType.DMA,
        ],
    )
    def k(tbl_hbm, idx_hbm, gkv_hbm, idx_v, buf0, buf1, s0, s1):
        wid = lax.axis_index("s") * nc + lax.axis_index("c")
        pltpu.sync_copy(idx_hbm.at[wid], idx_v)
        bufs = (buf0, buf1)
        sems = (s0, s1)
        pltpu.async_copy(tbl_hbm.at[idx_v.at[0]], buf0, s0)

        def body(i, carry):
            for t in range(2):
                c = 2 * i + t
                buf, sem = bufs[t], sems[t]
                nbuf, nsem = bufs[1 - t], sems[1 - t]
                nc_ = c + 1

                @pl.when(nc_ < nch)
                def _():
                    pltpu.async_copy(tbl_hbm.at[idx_v.at[nc_]], nbuf, nsem)

                pltpu.make_async_copy(tbl_hbm.at[idx_v.at[c]], buf, sem).wait()
                row0 = wid * per_w + c * ch
                pltpu.sync_copy(buf, gkv_hbm.at[pl.ds(row0, ch)])
            return carry

        lax.fori_loop(0, nch // 2, body, 0)

    return k(tbl2, idx3)


def _layernorm(x, g, b):
    m = jnp.mean(x, axis=1, keepdims=True)
    c = x - m
    v = jnp.mean(c * c, axis=1, keepdims=True)
    return c * lax.rsqrt(v + 1e-5) * g + b


def _stage_b_body(q_ref, gkv_ref, xyzq_ref, feat_ref,
                  fd1_ref, fd1b_ref, fd2_ref, fd2b_ref,
                  fg1_ref, fg1b_ref, fg2_ref, fg2b_ref,
                  proj_ref, n1g_ref, n1b_ref,
                  fc2_ref, fc2b_ref, n2g_ref, n2b_ref,
                  attn_ref, res_ref):
    kk = gkv_ref[:, :DM]
    vv = gkv_ref[:, DM:2 * DM]
    nx = gkv_ref[:, 2 * DM:2 * DM + 3]

    xq = xyzq_ref[0]
    rel = jnp.broadcast_to(xq[:, None, :], (QB, K, 3)).reshape(RB, 3) - nx
    p1 = jax.nn.relu(
        lax.dot_general(rel, fd1_ref[...], (((1,), (0,)), ((), ())))
        + fd1b_ref[...])
    pos = jnp.dot(p1, fd2_ref[...]) + fd2b_ref[...]

    qq = jnp.broadcast_to(q_ref[0][:, None, :], (QB, K, DM)).reshape(RB, DM)
    h = qq - kk + pos
    a = jax.nn.relu(jnp.dot(h, fg1_ref[...]) + fg1b_ref[...])
    a = jnp.dot(a, fg2_ref[...]) + fg2b_ref[...]
    a = a * jnp.float32(1.0 / math.sqrt(HD))
    a3 = a.reshape(QB, K, DM)
    m = jnp.max(a3, axis=1, keepdims=True)
    e = jnp.exp(a3 - m)
    s = jnp.sum(e, axis=1, keepdims=True)
    attn = e / s

    a_t = jnp.transpose(attn.reshape(QB, K * DM), (1, 0)).reshape(K, DM, QB)
    for hh in range(H):
        attn_ref[0, hh] = a_t[:, hh * HD:(hh + 1) * HD, :]

    w = attn * (vv + pos).reshape(QB, K, DM)
    resv = jnp.sum(w, axis=1)
    r1 = _layernorm(jnp.dot(resv, proj_ref[...]), n1g_ref[...], n1b_ref[...])
    r2 = jnp.dot(r1, fc2_ref[...]) + fc2b_ref[...]
    r2 = _layernorm(r2, n2g_ref[...], n2b_ref[...]) + feat_ref[0]
    res_ref[0] = jnp.transpose(r2, (1, 0))


def _stage_b(q, gkv, xyz, feats, wp):
    grid = (B, N // QB)
    nb = N // QB
    full = lambda r, c: pl.BlockSpec((r, c), lambda b, i: (0, 0))
    out = pl.pallas_call(
        _stage_b_body,
        grid=grid,
        in_specs=[
            pl.BlockSpec((1, QB, DM), lambda b, i: (b, i, 0)),
            pl.BlockSpec((RB, 3 * DM), lambda b, i: (b * nb + i, 0)),
            pl.BlockSpec((1, QB, 3), lambda b, i: (b, i, 0)),
            pl.BlockSpec((1, QB, DP), lambda b, i: (b, i, 0)),
            full(3, DM), full(1, DM), full(DM, DM), full(1, DM),
            full(DM, DM), full(1, DM), full(DM, DM), full(1, DM),
            full(DM, DM), full(1, DM), full(1, DM),
            full(DM, DP), full(1, DP), full(1, DP), full(1, DP),
        ],
        out_specs=[
            pl.BlockSpec((1, H, K, HD, QB), lambda b, i: (b, 0, 0, 0, i)),
            pl.BlockSpec((1, DP, QB), lambda b, i: (b, 0, i)),
        ],
        out_shape=[
            jax.ShapeDtypeStruct((B, H, K, HD, N), jnp.float32),
            jax.ShapeDtypeStruct((B, DP, N), jnp.float32),
        ],
    )(q, gkv, xyz, feats, *wp)
    return out


def kernel(xyz, features, params):
    p = params
    xyzt = xyz.transpose(0, 2, 1)
    q, tbl, knn = _stage_a(xyz, xyzt, features, p)

    info = plsc.get_sparse_core_info()
    nc, ns = info.num_cores, info.num_subcores
    nw = nc * ns
    tbl2 = tbl.reshape(B * N, 3 * DM)
    idx3 = knn.reshape(nw, (B * N * K) // (nw * 128), 128)
    gkv = _sc_gather(tbl2, idx3, nc, nw)

    z = jnp.zeros((HD, HD), jnp.float32)
    def bd(w):
        rows = []
        for i in range(H):
            rows.append(jnp.concatenate(
                [w if i == j else z for j in range(H)], axis=1))
        return jnp.concatenate(rows, axis=0)
    wp = (
        p['fd1_w'], p['fd1_b'].reshape(1, DM), p['fd2_w'],
        p['fd2_b'].reshape(1, DM),
        bd(p['fg1_w']), jnp.tile(p['fg1_b'], H).reshape(1, DM),
        bd(p['fg2_w']), jnp.tile(p['fg2_b'], H).reshape(1, DM),
        p['proj'], p['n1_g'].reshape(1, DM), p['n1_b'].reshape(1, DM),
        p['fc2_w'], p['fc2_b'].reshape(1, DP),
        p['n2_g'].reshape(1, DP), p['n2_b'].reshape(1, DP),
    )
    attn5, res_t = _stage_b(q, gkv, xyz, features, wp)
    attn = attn5.transpose(0, 1, 4, 2, 3).reshape(B * H, N, K, HD)
    res = res_t.transpose(0, 2, 1)
    return res, attn


# speedup vs baseline: 7.1242x; 7.1242x over previous
import functools
import math

import jax
import jax.numpy as jnp
from jax import lax
from jax.experimental import pallas as pl
from jax.experimental.pallas import tpu as pltpu
from jax.experimental.pallas import tpu_sc as plsc

B, N = 8, 2048
DP, DM, K, H = 64, 128, 16, 4
HD = DM // H

QA = 256
QB = 128
RB = QB * K


def _stage_a_body(xyzq_ref, xyzt_ref, feat_ref, fc1_ref, fc1b_ref,
                  wq_ref, wk_ref, wv_ref, q_ref, tbl_ref, knn_ref):
    b = pl.program_id(0)
    x = jnp.dot(feat_ref[0], fc1_ref[...]) + fc1b_ref[...]
    q_ref[0] = jnp.dot(x, wq_ref[...])
    xk = jnp.dot(x, wk_ref[...])
    xv = jnp.dot(x, wv_ref[...])

    xq = xyzq_ref[0]
    xt = xyzt_ref[0]
    tbl_ref[0] = jnp.concatenate(
        [xk, xv, xq, jnp.zeros((QA, DM - 3), jnp.float32)], axis=1)

    qn = jnp.sum(xq * xq, axis=1, keepdims=True)
    kn = jnp.sum(xt * xt, axis=0, keepdims=True)
    cross = lax.dot_general(xq, xt, (((1,), (0,)), ((), ())),
                            precision=lax.Precision.DEFAULT)
    d = qn + kn - 2.0 * cross

    iota = lax.broadcasted_iota(jnp.int32, (QA, N), 1)
    cols = []
    for j in range(K):
        m = jnp.min(d, axis=1, keepdims=True)
        idx = jnp.min(jnp.where(d == m, iota, N), axis=1, keepdims=True)
        cols.append(idx)
        d = jnp.where(iota == idx, jnp.float32(jnp.inf), d)
    knn_ref[0] = jnp.concatenate(cols, axis=1) + b * N


def _stage_a(xyz, xyzt, feats, p):
    grid = (B, N // QA)
    out = pl.pallas_call(
        _stage_a_body,
        grid=grid,
        in_specs=[
            pl.BlockSpec((1, QA, 3), lambda b, i: (b, i, 0)),
            pl.BlockSpec((1, 3, N), lambda b, i: (b, 0, 0)),
            pl.BlockSpec((1, QA, DP), lambda b, i: (b, i, 0)),
            pl.BlockSpec((DP, DM), lambda b, i: (0, 0)),
            pl.BlockSpec((1, DM), lambda b, i: (0, 0)),
            pl.BlockSpec((DM, DM), lambda b, i: (0, 0)),
            pl.BlockSpec((DM, DM), lambda b, i: (0, 0)),
            pl.BlockSpec((DM, DM), lambda b, i: (0, 0)),
        ],
        out_specs=[
            pl.BlockSpec((1, QA, DM), lambda b, i: (b, i, 0)),
            pl.BlockSpec((1, QA, 3 * DM), lambda b, i: (b, i, 0)),
            pl.BlockSpec((1, QA, K), lambda b, i: (b, i, 0)),
        ],
        out_shape=[
            jax.ShapeDtypeStruct((B, N, DM), jnp.float32),
            jax.ShapeDtypeStruct((B, N, 3 * DM), jnp.float32),
            jax.ShapeDtypeStruct((B, N, K), jnp.int32),
        ],
    )(xyz, xyzt, feats, p['fc1_w'], p['fc1_b'].reshape(1, DM),
      p['wq'], p['wk'], p['wv'])
    return out


def _sc_gather(tbl2, idx3, nc, nw):
    tot = B * N * K
    per_w = tot // nw
    ch = 128
    nch = per_w // ch

    mesh = plsc.VectorSubcoreMesh(core_axis_name="c", subcore_axis_name="s")

    @functools.partial(
        pl.kernel,
        mesh=mesh,
        out_type=jax.ShapeDtypeStruct((tot, 3 * DM), jnp.float32),
        scratch_types=[
            pltpu.VMEM((nch, ch), jnp.int32),
            pltpu.VMEM((ch, 3 * DM), jnp.float32),
            pltpu.VMEM((ch, 3 * DM), jnp.float32),
            pltpu.SemaphoreType.DMA,
            pltpu.SemaphoreType.DMA,
        ],
    )
    def k(tbl_hbm, idx_hbm, gkv_hbm, idx_v, buf0, buf1, s0, s1):
        wid = lax.axis_index("s") * nc + lax.axis_index("c")
        pltpu.sync_copy(idx_hbm.at[wid], idx_v)
        bufs = (buf0, buf1)
        sems = (s0, s1)
        pltpu.async_copy(tbl_hbm.at[idx_v.at[0]], buf0, s0)

        def body(i, carry):
            for t in range(2):
                c = 2 * i + t
                buf, sem = bufs[t], sems[t]
                nbuf, nsem = bufs[1 - t], sems[1 - t]
                nc_ = c + 1

                @pl.when(nc_ < nch)
                def _():
                    pltpu.async_copy(tbl_hbm.at[idx_v.at[nc_]], nbuf, nsem)

                pltpu.make_async_copy(tbl_hbm.at[idx_v.at[c]], buf, sem).wait()
                row0 = wid * per_w + c * ch
                pltpu.sync_copy(buf, gkv_hbm.at[pl.ds(row0, ch)])
            return carry

        lax.fori_loop(0, nch // 2, body, 0)

    return k(tbl2, idx3)


def _layernorm(x, g, b):
    m = jnp.mean(x, axis=1, keepdims=True)
    c = x - m
    v = jnp.mean(c * c, axis=1, keepdims=True)
    return c * lax.rsqrt(v + 1e-5) * g + b


def _stage_b_body(q_ref, gkv_ref, xyzq_ref, feat_ref,
                  fd1_ref, fd1b_ref, fd2_ref, fd2b_ref,
                  fg1_ref, fg1b_ref, fg2_ref, fg2b_ref,
                  proj_ref, n1g_ref, n1b_ref,
                  fc2_ref, fc2b_ref, n2g_ref, n2b_ref,
                  attn_ref, res_ref):
    kk = gkv_ref[:, :DM]
    vv = gkv_ref[:, DM:2 * DM]
    nx = gkv_ref[:, 2 * DM:2 * DM + 3]

    xq = xyzq_ref[0]
    rel = jnp.broadcast_to(xq[:, None, :], (QB, K, 3)).reshape(RB, 3) - nx
    p1 = jax.nn.relu(
        lax.dot_general(rel, fd1_ref[...], (((1,), (0,)), ((), ())))
        + fd1b_ref[...])
    pos = jnp.dot(p1, fd2_ref[...]) + fd2b_ref[...]

    qq = jnp.broadcast_to(q_ref[0][:, None, :], (QB, K, DM)).reshape(RB, DM)
    h = qq - kk + pos
    a = jax.nn.relu(jnp.dot(h, fg1_ref[...]) + fg1b_ref[...])
    a = jnp.dot(a, fg2_ref[...]) + fg2b_ref[...]
    a = a * jnp.float32(1.0 / math.sqrt(HD))
    a3 = a.reshape(QB, K, DM)
    m = jnp.max(a3, axis=1, keepdims=True)
    e = jnp.exp(a3 - m)
    s = jnp.sum(e, axis=1, keepdims=True)
    attn = e / s

    a_t = jnp.transpose(attn.reshape(QB, K * DM), (1, 0)).reshape(K, DM, QB)
    for hh in range(H):
        attn_ref[0, hh] = a_t[:, hh * HD:(hh + 1) * HD, :]

    w = attn * (vv + pos).reshape(QB, K, DM)
    resv = jnp.sum(w, axis=1)
    r1 = _layernorm(jnp.dot(resv, proj_ref[...]), n1g_ref[...], n1b_ref[...])
    r2 = jnp.dot(r1, fc2_ref[...]) + fc2b_ref[...]
    r2 = _layernorm(r2, n2g_ref[...], n2b_ref[...]) + feat_ref[0]
    res_ref[0] = jnp.transpose(r2, (1, 0))


def _stage_b(q, gkv, xyz, feats, wp):
    grid = (B, N // QB)
    nb = N // QB
    full = lambda r, c: pl.BlockSpec((r, c), lambda b, i: (0, 0))
    out = pl.pallas_call(
        _stage_b_body,
        grid=grid,
        in_specs=[
            pl.BlockSpec((1, QB, DM), lambda b, i: (b, i, 0)),
            pl.BlockSpec((RB, 3 * DM), lambda b, i: (b * nb + i, 0)),
            pl.BlockSpec((1, QB, 3), lambda b, i: (b, i, 0)),
            pl.BlockSpec((1, QB, DP), lambda b, i: (b, i, 0)),
            full(3, DM), full(1, DM), full(DM, DM), full(1, DM),
            full(DM, DM), full(1, DM), full(DM, DM), full(1, DM),
            full(DM, DM), full(1, DM), full(1, DM),
            full(DM, DP), full(1, DP), full(1, DP), full(1, DP),
        ],
        out_specs=[
            pl.BlockSpec((1, H, K, HD, QB), lambda b, i: (b, 0, 0, 0, i)),
            pl.BlockSpec((1, DP, QB), lambda b, i: (b, 0, i)),
        ],
        out_shape=[
            jax.ShapeDtypeStruct((B, H, K, HD, N), jnp.float32),
            jax.ShapeDtypeStruct((B, DP, N), jnp.float32),
        ],
    )(q, gkv, xyz, feats, *wp)
    return out


def kernel(xyz, features, params):
    p = params
    xyzt = xyz.transpose(0, 2, 1)
    q, tbl, knn = _stage_a(xyz, xyzt, features, p)

    info = plsc.get_sparse_core_info()
    nc, ns = info.num_cores, info.num_subcores
    nw = nc * ns
    tbl2 = tbl.reshape(B * N, 3 * DM)
    idx3 = knn.reshape(nw, (B * N * K) // (nw * 128), 128)
    gkv = _sc_gather(tbl2, idx3, nc, nw)

    z = jnp.zeros((HD, HD), jnp.float32)
    def bd(w):
        rows = []
        for i in range(H):
            rows.append(jnp.concatenate(
                [w if i == j else z for j in range(H)], axis=1))
        return jnp.concatenate(rows, axis=0)
    wp = (
        p['fd1_w'], p['fd1_b'].reshape(1, DM), p['fd2_w'],
        p['fd2_b'].reshape(1, DM),
        bd(p['fg1_w']), jnp.tile(p['fg1_b'], H).reshape(1, DM),
        bd(p['fg2_w']), jnp.tile(p['fg2_b'], H).reshape(1, DM),
        p['proj'], p['n1_g'].reshape(1, DM), p['n1_b'].reshape(1, DM),
        p['fc2_w'], p['fc2_b'].reshape(1, DP),
        p['n2_g'].reshape(1, DP), p['n2_b'].reshape(1, DP),
    )
    attn5, res_t = _stage_b(q, gkv, xyz, features, wp)
    attn = attn5.transpose(0, 1, 4, 2, 3).reshape(B * H, N, K, HD)
    res = res_t.transpose(0, 2, 1)
    return res, attn


# speedup vs baseline: 7.8679x; 1.1044x over previous
import functools
import math

import jax
import jax.numpy as jnp
from jax import lax
from jax.experimental import pallas as pl
from jax.experimental.pallas import tpu as pltpu
from jax.experimental.pallas import tpu_sc as plsc

B, N = 8, 2048
DP, DM, K, H = 64, 128, 16, 4
HD = DM // H

QA = 256
QB = 256
RB = QB * K


def _stage_a_body(xyzq_ref, xyzt_ref, feat_ref, fc1_ref, fc1b_ref,
                  wq_ref, wk_ref, wv_ref, q_ref, tbl_ref, knn_ref):
    b = pl.program_id(0)
    x = jnp.dot(feat_ref[0], fc1_ref[...]) + fc1b_ref[...]
    q_ref[0] = jnp.dot(x, wq_ref[...])
    xk = jnp.dot(x, wk_ref[...])
    xv = jnp.dot(x, wv_ref[...])

    xq = xyzq_ref[0]
    xt = xyzt_ref[0]
    tbl_ref[0] = jnp.concatenate(
        [xk, xv, xq, jnp.zeros((QA, DM - 3), jnp.float32)], axis=1)

    qn = jnp.sum(xq * xq, axis=1, keepdims=True)
    kn = jnp.sum(xt * xt, axis=0, keepdims=True)
    cross = lax.dot_general(xq, xt, (((1,), (0,)), ((), ())),
                            precision=lax.Precision.DEFAULT)
    d = qn + kn - 2.0 * cross

    iota_f = lax.broadcasted_iota(jnp.int32, (QA, N), 1).astype(jnp.float32)
    cols = []
    for j in range(K):
        cv, ci = d, iota_f
        w = N // 2
        while w >= 128:
            lo_v, hi_v = cv[:, :w], cv[:, w:]
            lo_i, hi_i = ci[:, :w], ci[:, w:]
            take_hi = hi_v < lo_v
            cv = jnp.where(take_hi, hi_v, lo_v)
            ci = jnp.where(take_hi, hi_i, lo_i)
            w //= 2
        m = jnp.min(cv, axis=1, keepdims=True)
        idxf = jnp.min(jnp.where(cv == m, ci, jnp.float32(N)),
                       axis=1, keepdims=True)
        cols.append(idxf)
        d = jnp.where(iota_f == idxf, jnp.float32(jnp.inf), d)
    knn_ref[0] = (jnp.concatenate(cols, axis=1).astype(jnp.int32) + b * N)


def _stage_a(xyz, xyzt, feats, p):
    grid = (B, N // QA)
    out = pl.pallas_call(
        _stage_a_body,
        grid=grid,
        in_specs=[
            pl.BlockSpec((1, QA, 3), lambda b, i: (b, i, 0)),
            pl.BlockSpec((1, 3, N), lambda b, i: (b, 0, 0)),
            pl.BlockSpec((1, QA, DP), lambda b, i: (b, i, 0)),
            pl.BlockSpec((DP, DM), lambda b, i: (0, 0)),
            pl.BlockSpec((1, DM), lambda b, i: (0, 0)),
            pl.BlockSpec((DM, DM), lambda b, i: (0, 0)),
            pl.BlockSpec((DM, DM), lambda b, i: (0, 0)),
            pl.BlockSpec((DM, DM), lambda b, i: (0, 0)),
        ],
        out_specs=[
            pl.BlockSpec((1, QA, DM), lambda b, i: (b, i, 0)),
            pl.BlockSpec((1, QA, 3 * DM), lambda b, i: (b, i, 0)),
            pl.BlockSpec((1, QA, K), lambda b, i: (b, i, 0)),
        ],
        out_shape=[
            jax.ShapeDtypeStruct((B, N, DM), jnp.float32),
            jax.ShapeDtypeStruct((B, N, 3 * DM), jnp.float32),
            jax.ShapeDtypeStruct((B, N, K), jnp.int32),
        ],
    )(xyz, xyzt, feats, p['fc1_w'], p['fc1_b'].reshape(1, DM),
      p['wq'], p['wk'], p['wv'])
    return out


def _sc_gather(tbl2, idx3, nc, nw):
    tot = B * N * K
    per_w = tot // nw
    ch = 128
    nch = per_w // ch

    mesh = plsc.VectorSubcoreMesh(core_axis_name="c", subcore_axis_name="s")

    @functools.partial(
        pl.kernel,
        mesh=mesh,
        out_type=jax.ShapeDtypeStruct((tot, 3 * DM), jnp.float32),
        scratch_types=[
            pltpu.VMEM((nch, ch), jnp.int32),
            pltpu.VMEM((ch, 3 * DM), jnp.float32),
            pltpu.VMEM((ch, 3 * DM), jnp.float32),
            pltpu.SemaphoreType.DMA,
            pltpu.SemaphoreType.DMA,
        ],
    )
    def k(tbl_hbm, idx_hbm, gkv_hbm, idx_v, buf0, buf1, s0, s1):
        wid = lax.axis_index("s") * nc + lax.axis_index("c")
        pltpu.sync_copy(idx_hbm.at[wid], idx_v)
        bufs = (buf0, buf1)
        sems = (s0, s1)
        pltpu.async_copy(tbl_hbm.at[idx_v.at[0]], buf0, s0)

        def body(i, carry):
            for t in range(2):
                c = 2 * i + t
                buf, sem = bufs[t], sems[t]
                nbuf, nsem = bufs[1 - t], sems[1 - t]
                nc_ = c + 1

                @pl.when(nc_ < nch)
                def _():
                    pltpu.async_copy(tbl_hbm.at[idx_v.at[nc_]], nbuf, nsem)

                pltpu.make_async_copy(tbl_hbm.at[idx_v.at[c]], buf, sem).wait()
                row0 = wid * per_w + c * ch
                pltpu.sync_copy(buf, gkv_hbm.at[pl.ds(row0, ch)])
            return carry

        lax.fori_loop(0, nch // 2, body, 0)

    return k(tbl2, idx3)


def _layernorm(x, g, b):
    m = jnp.mean(x, axis=1, keepdims=True)
    c = x - m
    v = jnp.mean(c * c, axis=1, keepdims=True)
    return c * lax.rsqrt(v + 1e-5) * g + b


def _stage_b_body(q_ref, gkv_ref, xyzq_ref, feat_ref,
                  fd1_ref, fd1b_ref, fd2_ref, fd2b_ref,
                  fg1_ref, fg1b_ref, fg2_ref, fg2b_ref,
                  proj_ref, n1g_ref, n1b_ref,
                  fc2_ref, fc2b_ref, n2g_ref, n2b_ref,
                  attn_ref, res_ref):
    kk = gkv_ref[:, :DM]
    vv = gkv_ref[:, DM:2 * DM]
    nx = gkv_ref[:, 2 * DM:2 * DM + 3]

    xq = xyzq_ref[0]
    rel = jnp.broadcast_to(xq[:, None, :], (QB, K, 3)).reshape(RB, 3) - nx
    p1 = jax.nn.relu(
        lax.dot_general(rel, fd1_ref[...], (((1,), (0,)), ((), ())))
        + fd1b_ref[...])
    pos = jnp.dot(p1, fd2_ref[...]) + fd2b_ref[...]

    qq = jnp.broadcast_to(q_ref[0][:, None, :], (QB, K, DM)).reshape(RB, DM)
    h = qq - kk + pos
    a = jax.nn.relu(jnp.dot(h, fg1_ref[...]) + fg1b_ref[...])
    a = jnp.dot(a, fg2_ref[...]) + fg2b_ref[...]
    a = a * jnp.float32(1.0 / math.sqrt(HD))
    a3 = a.reshape(QB, K, DM)
    m = jnp.max(a3, axis=1, keepdims=True)
    e = jnp.exp(a3 - m)
    s = jnp.sum(e, axis=1, keepdims=True)
    attn = e / s

    a_t = jnp.transpose(attn.reshape(QB, K * DM), (1, 0)).reshape(K, DM, QB)
    for hh in range(H):
        attn_ref[0, hh] = a_t[:, hh * HD:(hh + 1) * HD, :]

    w = attn * (vv + pos).reshape(QB, K, DM)
    resv = jnp.sum(w, axis=1)
    r1 = _layernorm(jnp.dot(resv, proj_ref[...]), n1g_ref[...], n1b_ref[...])
    r2 = jnp.dot(r1, fc2_ref[...]) + fc2b_ref[...]
    r2 = _layernorm(r2, n2g_ref[...], n2b_ref[...]) + feat_ref[0]
    res_ref[0] = jnp.transpose(r2, (1, 0))


def _stage_b(q, gkv, xyz, feats, wp):
    grid = (B, N // QB)
    nb = N // QB
    full = lambda r, c: pl.BlockSpec((r, c), lambda b, i: (0, 0))
    out = pl.pallas_call(
        _stage_b_body,
        grid=grid,
        in_specs=[
            pl.BlockSpec((1, QB, DM), lambda b, i: (b, i, 0)),
            pl.BlockSpec((RB, 3 * DM), lambda b, i: (b * nb + i, 0)),
            pl.BlockSpec((1, QB, 3), lambda b, i: (b, i, 0)),
            pl.BlockSpec((1, QB, DP), lambda b, i: (b, i, 0)),
            full(3, DM), full(1, DM), full(DM, DM), full(1, DM),
            full(DM, DM), full(1, DM), full(DM, DM), full(1, DM),
            full(DM, DM), full(1, DM), full(1, DM),
            full(DM, DP), full(1, DP), full(1, DP), full(1, DP),
        ],
        out_specs=[
            pl.BlockSpec((1, H, K, HD, QB), lambda b, i: (b, 0, 0, 0, i)),
            pl.BlockSpec((1, DP, QB), lambda b, i: (b, 0, i)),
        ],
        out_shape=[
            jax.ShapeDtypeStruct((B, H, K, HD, N), jnp.float32),
            jax.ShapeDtypeStruct((B, DP, N), jnp.float32),
        ],
    )(q, gkv, xyz, feats, *wp)
    return out


def kernel(xyz, features, params):
    p = params
    xyzt = xyz.transpose(0, 2, 1)
    q, tbl, knn = _stage_a(xyz, xyzt, features, p)

    info = plsc.get_sparse_core_info()
    nc, ns = info.num_cores, info.num_subcores
    nw = nc * ns
    tbl2 = tbl.reshape(B * N, 3 * DM)
    idx3 = knn.reshape(nw, (B * N * K) // (nw * 128), 128)
    gkv = _sc_gather(tbl2, idx3, nc, nw)

    z = jnp.zeros((HD, HD), jnp.float32)
    def bd(w):
        rows = []
        for i in range(H):
            rows.append(jnp.concatenate(
                [w if i == j else z for j in range(H)], axis=1))
        return jnp.concatenate(rows, axis=0)
    wp = (
        p['fd1_w'], p['fd1_b'].reshape(1, DM), p['fd2_w'],
        p['fd2_b'].reshape(1, DM),
        bd(p['fg1_w']), jnp.tile(p['fg1_b'], H).reshape(1, DM),
        bd(p['fg2_w']), jnp.tile(p['fg2_b'], H).reshape(1, DM),
        p['proj'], p['n1_g'].reshape(1, DM), p['n1_b'].reshape(1, DM),
        p['fc2_w'], p['fc2_b'].reshape(1, DP),
        p['n2_g'].reshape(1, DP), p['n2_b'].reshape(1, DP),
    )
    attn5, res_t = _stage_b(q, gkv, xyz, features, wp)
    attn = attn5.transpose(0, 1, 4, 2, 3).reshape(B * H, N, K, HD)
    res = res_t.transpose(0, 2, 1)
    return res, attn


# speedup vs baseline: 8.5289x; 1.0840x over previous
import functools
import math

import jax
import jax.numpy as jnp
from jax import lax
from jax.experimental import pallas as pl
from jax.experimental.pallas import tpu as pltpu
from jax.experimental.pallas import tpu_sc as plsc

B, N = 8, 2048
DP, DM, K, H = 64, 128, 16, 4
HD = DM // H

QA = 256
QB = 256
RB = QB * K


def _stage_a_body(xyzq_ref, xyzt_ref, feat_ref, fc1_ref, fc1b_ref,
                  wq_ref, wk_ref, wv_ref, q_ref, tbl_ref, knn_ref):
    b = pl.program_id(0)
    x = jnp.dot(feat_ref[0], fc1_ref[...]) + fc1b_ref[...]
    q_ref[0] = jnp.dot(x, wq_ref[...])
    xk = jnp.dot(x, wk_ref[...])
    xv = jnp.dot(x, wv_ref[...])

    xq = xyzq_ref[0]
    xt = xyzt_ref[0]
    k16 = lax.bitcast_convert_type(xk.astype(jnp.bfloat16), jnp.uint16)
    v16 = lax.bitcast_convert_type(xv.astype(jnp.bfloat16), jnp.uint16)
    kv = (v16.astype(jnp.uint32) << 16) | k16.astype(jnp.uint32)
    tbl_ref[0] = jnp.concatenate(
        [lax.bitcast_convert_type(kv, jnp.float32), xq,
         jnp.zeros((QA, DM - 3), jnp.float32)], axis=1)

    qn = jnp.sum(xq * xq, axis=1, keepdims=True)
    kn = jnp.sum(xt * xt, axis=0, keepdims=True)
    cross = lax.dot_general(xq, xt, (((1,), (0,)), ((), ())),
                            precision=lax.Precision.DEFAULT)
    d = qn + kn - 2.0 * cross

    iota_f = lax.broadcasted_iota(jnp.int32, (QA, N), 1).astype(jnp.float32)
    cols = []
    for j in range(K):
        cv, ci = d, iota_f
        w = N // 2
        while w >= 128:
            lo_v, hi_v = cv[:, :w], cv[:, w:]
            lo_i, hi_i = ci[:, :w], ci[:, w:]
            take_hi = hi_v < lo_v
            cv = jnp.where(take_hi, hi_v, lo_v)
            ci = jnp.where(take_hi, hi_i, lo_i)
            w //= 2
        m = jnp.min(cv, axis=1, keepdims=True)
        idxf = jnp.min(jnp.where(cv == m, ci, jnp.float32(N)),
                       axis=1, keepdims=True)
        cols.append(idxf)
        d = jnp.where(iota_f == idxf, jnp.float32(jnp.inf), d)
    knn_ref[0] = (jnp.concatenate(cols, axis=1).astype(jnp.int32) + b * N)


def _stage_a(xyz, xyzt, feats, p):
    grid = (B, N // QA)
    out = pl.pallas_call(
        _stage_a_body,
        grid=grid,
        in_specs=[
            pl.BlockSpec((1, QA, 3), lambda b, i: (b, i, 0)),
            pl.BlockSpec((1, 3, N), lambda b, i: (b, 0, 0)),
            pl.BlockSpec((1, QA, DP), lambda b, i: (b, i, 0)),
            pl.BlockSpec((DP, DM), lambda b, i: (0, 0)),
            pl.BlockSpec((1, DM), lambda b, i: (0, 0)),
            pl.BlockSpec((DM, DM), lambda b, i: (0, 0)),
            pl.BlockSpec((DM, DM), lambda b, i: (0, 0)),
            pl.BlockSpec((DM, DM), lambda b, i: (0, 0)),
        ],
        out_specs=[
            pl.BlockSpec((1, QA, DM), lambda b, i: (b, i, 0)),
            pl.BlockSpec((1, QA, 2 * DM), lambda b, i: (b, i, 0)),
            pl.BlockSpec((1, QA, K), lambda b, i: (b, i, 0)),
        ],
        out_shape=[
            jax.ShapeDtypeStruct((B, N, DM), jnp.float32),
            jax.ShapeDtypeStruct((B, N, 2 * DM), jnp.float32),
            jax.ShapeDtypeStruct((B, N, K), jnp.int32),
        ],
    )(xyz, xyzt, feats, p['fc1_w'], p['fc1_b'].reshape(1, DM),
      p['wq'], p['wk'], p['wv'])
    return out


def _sc_gather(tbl2, idx3, nc, nw):
    tot = B * N * K
    per_w = tot // nw
    ch = 128
    nch = per_w // ch

    mesh = plsc.VectorSubcoreMesh(core_axis_name="c", subcore_axis_name="s")

    @functools.partial(
        pl.kernel,
        mesh=mesh,
        out_type=jax.ShapeDtypeStruct((tot, 2 * DM), jnp.float32),
        scratch_types=[
            pltpu.VMEM((nch, ch), jnp.int32),
            pltpu.VMEM((ch, 2 * DM), jnp.float32),
            pltpu.VMEM((ch, 2 * DM), jnp.float32),
            pltpu.SemaphoreType.DMA,
            pltpu.SemaphoreType.DMA,
        ],
    )
    def k(tbl_hbm, idx_hbm, gkv_hbm, idx_v, buf0, buf1, s0, s1):
        wid = lax.axis_index("s") * nc + lax.axis_index("c")
        pltpu.sync_copy(idx_hbm.at[wid], idx_v)
        bufs = (buf0, buf1)
        sems = (s0, s1)
        pltpu.async_copy(tbl_hbm.at[idx_v.at[0]], buf0, s0)

        def body(i, carry):
            for t in range(2):
                c = 2 * i + t
                buf, sem = bufs[t], sems[t]
                nbuf, nsem = bufs[1 - t], sems[1 - t]
                nc_ = c + 1

                @pl.when(nc_ < nch)
                def _():
                    pltpu.async_copy(tbl_hbm.at[idx_v.at[nc_]], nbuf, nsem)

                pltpu.make_async_copy(tbl_hbm.at[idx_v.at[c]], buf, sem).wait()
                row0 = wid * per_w + c * ch
                pltpu.sync_copy(buf, gkv_hbm.at[pl.ds(row0, ch)])
            return carry

        lax.fori_loop(0, nch // 2, body, 0)

    return k(tbl2, idx3)


def _layernorm(x, g, b):
    m = jnp.mean(x, axis=1, keepdims=True)
    c = x - m
    v = jnp.mean(c * c, axis=1, keepdims=True)
    return c * lax.rsqrt(v + 1e-5) * g + b


def _stage_b_body(q_ref, gkv_ref, xyzq_ref, feat_ref,
                  fd1_ref, fd1b_ref, fd2_ref, fd2b_ref,
                  fg1_ref, fg1b_ref, fg2_ref, fg2b_ref,
                  proj_ref, n1g_ref, n1b_ref,
                  fc2_ref, fc2b_ref, n2g_ref, n2b_ref,
                  attn_ref, res_ref):
    u = lax.bitcast_convert_type(gkv_ref[:, :DM], jnp.uint32)
    kk = lax.bitcast_convert_type(u << 16, jnp.float32)
    vv = lax.bitcast_convert_type(u & jnp.uint32(0xFFFF0000), jnp.float32)
    nx = gkv_ref[:, DM:DM + 3]

    xq = xyzq_ref[0]
    rel = jnp.broadcast_to(xq[:, None, :], (QB, K, 3)).reshape(RB, 3) - nx
    p1 = jax.nn.relu(
        lax.dot_general(rel, fd1_ref[...], (((1,), (0,)), ((), ())))
        + fd1b_ref[...])
    pos = jnp.dot(p1, fd2_ref[...]) + fd2b_ref[...]

    qq = jnp.broadcast_to(q_ref[0][:, None, :], (QB, K, DM)).reshape(RB, DM)
    h = qq - kk + pos
    a = jax.nn.relu(jnp.dot(h, fg1_ref[...]) + fg1b_ref[...])
    a = jnp.dot(a, fg2_ref[...]) + fg2b_ref[...]
    a = a * jnp.float32(1.0 / math.sqrt(HD))
    a3 = a.reshape(QB, K, DM)
    m = jnp.max(a3, axis=1, keepdims=True)
    e = jnp.exp(a3 - m)
    s = jnp.sum(e, axis=1, keepdims=True)
    attn = e / s

    a_t = jnp.transpose(attn.reshape(QB, K * DM), (1, 0)).reshape(K, DM, QB)
    for hh in range(H):
        attn_ref[0, hh] = a_t[:, hh * HD:(hh + 1) * HD, :]

    w = attn * (vv + pos).reshape(QB, K, DM)
    resv = jnp.sum(w, axis=1)
    r1 = _layernorm(jnp.dot(resv, proj_ref[...]), n1g_ref[...], n1b_ref[...])
    r2 = jnp.dot(r1, fc2_ref[...]) + fc2b_ref[...]
    r2 = _layernorm(r2, n2g_ref[...], n2b_ref[...]) + feat_ref[0]
    res_ref[0] = jnp.transpose(r2, (1, 0))


def _stage_b(q, gkv, xyz, feats, wp):
    grid = (B, N // QB)
    nb = N // QB
    full = lambda r, c: pl.BlockSpec((r, c), lambda b, i: (0, 0))
    out = pl.pallas_call(
        _stage_b_body,
        grid=grid,
        in_specs=[
            pl.BlockSpec((1, QB, DM), lambda b, i: (b, i, 0)),
            pl.BlockSpec((RB, 2 * DM), lambda b, i: (b * nb + i, 0)),
            pl.BlockSpec((1, QB, 3), lambda b, i: (b, i, 0)),
            pl.BlockSpec((1, QB, DP), lambda b, i: (b, i, 0)),
            full(3, DM), full(1, DM), full(DM, DM), full(1, DM),
            full(DM, DM), full(1, DM), full(DM, DM), full(1, DM),
            full(DM, DM), full(1, DM), full(1, DM),
            full(DM, DP), full(1, DP), full(1, DP), full(1, DP),
        ],
        out_specs=[
            pl.BlockSpec((1, H, K, HD, QB), lambda b, i: (b, 0, 0, 0, i)),
            pl.BlockSpec((1, DP, QB), lambda b, i: (b, 0, i)),
        ],
        out_shape=[
            jax.ShapeDtypeStruct((B, H, K, HD, N), jnp.float32),
            jax.ShapeDtypeStruct((B, DP, N), jnp.float32),
        ],
    )(q, gkv, xyz, feats, *wp)
    return out


def kernel(xyz, features, params):
    p = params
    xyzt = xyz.transpose(0, 2, 1)
    q, tbl, knn = _stage_a(xyz, xyzt, features, p)

    info = plsc.get_sparse_core_info()
    nc, ns = info.num_cores, info.num_subcores
    nw = nc * ns
    tbl2 = tbl.reshape(B * N, 2 * DM)
    idx3 = knn.reshape(nw, (B * N * K) // (nw * 128), 128)
    gkv = _sc_gather(tbl2, idx3, nc, nw)

    z = jnp.zeros((HD, HD), jnp.float32)
    def bd(w):
        rows = []
        for i in range(H):
            rows.append(jnp.concatenate(
                [w if i == j else z for j in range(H)], axis=1))
        return jnp.concatenate(rows, axis=0)
    wp = (
        p['fd1_w'], p['fd1_b'].reshape(1, DM), p['fd2_w'],
        p['fd2_b'].reshape(1, DM),
        bd(p['fg1_w']), jnp.tile(p['fg1_b'], H).reshape(1, DM),
        bd(p['fg2_w']), jnp.tile(p['fg2_b'], H).reshape(1, DM),
        p['proj'], p['n1_g'].reshape(1, DM), p['n1_b'].reshape(1, DM),
        p['fc2_w'], p['fc2_b'].reshape(1, DP),
        p['n2_g'].reshape(1, DP), p['n2_b'].reshape(1, DP),
    )
    attn5, res_t = _stage_b(q, gkv, xyz, features, wp)
    attn = attn5.transpose(0, 1, 4, 2, 3).reshape(B * H, N, K, HD)
    res = res_t.transpose(0, 2, 1)
    return res, attn


# speedup vs baseline: 8.8707x; 1.0401x over previous
import functools
import math

import jax
import jax.numpy as jnp
from jax import lax
from jax.experimental import pallas as pl
from jax.experimental.pallas import tpu as pltpu
from jax.experimental.pallas import tpu_sc as plsc

B, N = 8, 2048
DP, DM, K, H = 64, 128, 16, 4
HD = DM // H

QA = 512
QB = 256
RB = QB * K


def _stage_a_body(xyzq_ref, xyzt_ref, feat_ref, fc1_ref, fc1b_ref,
                  wq_ref, wk_ref, wv_ref, q_ref, tbl_ref, knn_ref):
    b = pl.program_id(0)
    x = jnp.dot(feat_ref[0], fc1_ref[...]) + fc1b_ref[...]
    q_ref[0] = jnp.dot(x, wq_ref[...])
    xk = jnp.dot(x, wk_ref[...])
    xv = jnp.dot(x, wv_ref[...])

    xq = xyzq_ref[0]
    xt = xyzt_ref[0]
    k16 = lax.bitcast_convert_type(xk.astype(jnp.bfloat16), jnp.uint16)
    v16 = lax.bitcast_convert_type(xv.astype(jnp.bfloat16), jnp.uint16)
    kv = (v16.astype(jnp.uint32) << 16) | k16.astype(jnp.uint32)
    tbl_ref[0] = jnp.concatenate(
        [lax.bitcast_convert_type(kv, jnp.float32), xq,
         jnp.zeros((QA, DM - 3), jnp.float32)], axis=1)

    qn = jnp.sum(xq * xq, axis=1, keepdims=True)
    kn = jnp.sum(xt * xt, axis=0, keepdims=True)
    cross = lax.dot_general(xq, xt, (((1,), (0,)), ((), ())),
                            precision=lax.Precision.DEFAULT)
    d = qn + kn - 2.0 * cross

    iota_f = lax.broadcasted_iota(jnp.int32, (QA, N), 1).astype(jnp.float32)
    cols = []
    for j in range(K):
        cv, ci = d, iota_f
        w = N // 2
        while w >= 128:
            lo_v, hi_v = cv[:, :w], cv[:, w:]
            lo_i, hi_i = ci[:, :w], ci[:, w:]
            take_hi = hi_v < lo_v
            cv = jnp.where(take_hi, hi_v, lo_v)
            ci = jnp.where(take_hi, hi_i, lo_i)
            w //= 2
        m = jnp.min(cv, axis=1, keepdims=True)
        idxf = jnp.min(jnp.where(cv == m, ci, jnp.float32(N)),
                       axis=1, keepdims=True)
        cols.append(idxf)
        d = jnp.where(iota_f == idxf, jnp.float32(jnp.inf), d)
    knn_ref[0] = (jnp.concatenate(cols, axis=1).astype(jnp.int32) + b * N)


def _stage_a(xyz, xyzt, feats, p):
    grid = (B, N // QA)
    out = pl.pallas_call(
        _stage_a_body,
        grid=grid,
        in_specs=[
            pl.BlockSpec((1, QA, 3), lambda b, i: (b, i, 0)),
            pl.BlockSpec((1, 3, N), lambda b, i: (b, 0, 0)),
            pl.BlockSpec((1, QA, DP), lambda b, i: (b, i, 0)),
            pl.BlockSpec((DP, DM), lambda b, i: (0, 0)),
            pl.BlockSpec((1, DM), lambda b, i: (0, 0)),
            pl.BlockSpec((DM, DM), lambda b, i: (0, 0)),
            pl.BlockSpec((DM, DM), lambda b, i: (0, 0)),
            pl.BlockSpec((DM, DM), lambda b, i: (0, 0)),
        ],
        out_specs=[
            pl.BlockSpec((1, QA, DM), lambda b, i: (b, i, 0)),
            pl.BlockSpec((1, QA, 2 * DM), lambda b, i: (b, i, 0)),
            pl.BlockSpec((1, QA, K), lambda b, i: (b, i, 0)),
        ],
        out_shape=[
            jax.ShapeDtypeStruct((B, N, DM), jnp.float32),
            jax.ShapeDtypeStruct((B, N, 2 * DM), jnp.float32),
            jax.ShapeDtypeStruct((B, N, K), jnp.int32),
        ],
    )(xyz, xyzt, feats, p['fc1_w'], p['fc1_b'].reshape(1, DM),
      p['wq'], p['wk'], p['wv'])
    return out


def _sc_gather(tbl2, idx3, nc, nw):
    tot = B * N * K
    per_w = tot // nw
    ch = 128
    nch = per_w // ch

    mesh = plsc.VectorSubcoreMesh(core_axis_name="c", subcore_axis_name="s")

    @functools.partial(
        pl.kernel,
        mesh=mesh,
        out_type=jax.ShapeDtypeStruct((tot, 2 * DM), jnp.float32),
        scratch_types=[
            pltpu.VMEM((nch, ch), jnp.int32),
            pltpu.VMEM((ch, 2 * DM), jnp.float32),
            pltpu.VMEM((ch, 2 * DM), jnp.float32),
            pltpu.SemaphoreType.DMA,
            pltpu.SemaphoreType.DMA,
        ],
    )
    def k(tbl_hbm, idx_hbm, gkv_hbm, idx_v, buf0, buf1, s0, s1):
        wid = lax.axis_index("s") * nc + lax.axis_index("c")
        pltpu.sync_copy(idx_hbm.at[wid], idx_v)
        bufs = (buf0, buf1)
        sems = (s0, s1)
        pltpu.async_copy(tbl_hbm.at[idx_v.at[0]], buf0, s0)

        def body(i, carry):
            for t in range(2):
                c = 2 * i + t
                buf, sem = bufs[t], sems[t]
                nbuf, nsem = bufs[1 - t], sems[1 - t]
                nc_ = c + 1

                @pl.when(nc_ < nch)
                def _():
                    pltpu.async_copy(tbl_hbm.at[idx_v.at[nc_]], nbuf, nsem)

                pltpu.make_async_copy(tbl_hbm.at[idx_v.at[c]], buf, sem).wait()
                row0 = wid * per_w + c * ch
                pltpu.sync_copy(buf, gkv_hbm.at[pl.ds(row0, ch)])
            return carry

        lax.fori_loop(0, nch // 2, body, 0)

    return k(tbl2, idx3)


def _layernorm(x, g, b):
    m = jnp.mean(x, axis=1, keepdims=True)
    c = x - m
    v = jnp.mean(c * c, axis=1, keepdims=True)
    return c * lax.rsqrt(v + 1e-5) * g + b


def _stage_b_body(q_ref, gkv_ref, xyzq_ref, feat_ref,
                  fd1_ref, fd1b_ref, fd2_ref, fd2b_ref,
                  fg1_ref, fg1b_ref, fg2_ref, fg2b_ref,
                  proj_ref, n1g_ref, n1b_ref,
                  fc2_ref, fc2b_ref, n2g_ref, n2b_ref,
                  attn_ref, res_ref):
    u = lax.bitcast_convert_type(gkv_ref[:, :DM], jnp.uint32)
    kk = lax.bitcast_convert_type(u << 16, jnp.float32)
    vv = lax.bitcast_convert_type(u & jnp.uint32(0xFFFF0000), jnp.float32)
    nx = gkv_ref[:, DM:DM + 3]

    xq = xyzq_ref[0]
    rel = jnp.broadcast_to(xq[:, None, :], (QB, K, 3)).reshape(RB, 3) - nx
    p1 = jax.nn.relu(
        lax.dot_general(rel, fd1_ref[...], (((1,), (0,)), ((), ())))
        + fd1b_ref[...])
    pos = jnp.dot(p1, fd2_ref[...]) + fd2b_ref[...]

    qq = jnp.broadcast_to(q_ref[0][:, None, :], (QB, K, DM)).reshape(RB, DM)
    h = qq - kk + pos
    a = jax.nn.relu(jnp.dot(h, fg1_ref[...]) + fg1b_ref[...])
    a = jnp.dot(a, fg2_ref[...]) + fg2b_ref[...]
    a = a * jnp.float32(1.0 / math.sqrt(HD))
    a3 = a.reshape(QB, K, DM)
    m = jnp.max(a3, axis=1, keepdims=True)
    e = jnp.exp(a3 - m)
    s = jnp.sum(e, axis=1, keepdims=True)
    attn = e / s

    a_t = jnp.transpose(attn.reshape(QB, K * DM), (1, 0)).reshape(K, DM, QB)
    for hh in range(H):
        attn_ref[0, hh] = a_t[:, hh * HD:(hh + 1) * HD, :]

    w = attn * (vv + pos).reshape(QB, K, DM)
    resv = jnp.sum(w, axis=1)
    r1 = _layernorm(jnp.dot(resv, proj_ref[...]), n1g_ref[...], n1b_ref[...])
    r2 = jnp.dot(r1, fc2_ref[...]) + fc2b_ref[...]
    r2 = _layernorm(r2, n2g_ref[...], n2b_ref[...]) + feat_ref[0]
    res_ref[0] = jnp.transpose(r2, (1, 0))


def _stage_b(q, gkv, xyz, feats, wp):
    grid = (B, N // QB)
    nb = N // QB
    full = lambda r, c: pl.BlockSpec((r, c), lambda b, i: (0, 0))
    out = pl.pallas_call(
        _stage_b_body,
        grid=grid,
        in_specs=[
            pl.BlockSpec((1, QB, DM), lambda b, i: (b, i, 0)),
            pl.BlockSpec((RB, 2 * DM), lambda b, i: (b * nb + i, 0)),
            pl.BlockSpec((1, QB, 3), lambda b, i: (b, i, 0)),
            pl.BlockSpec((1, QB, DP), lambda b, i: (b, i, 0)),
            full(3, DM), full(1, DM), full(DM, DM), full(1, DM),
            full(DM, DM), full(1, DM), full(DM, DM), full(1, DM),
            full(DM, DM), full(1, DM), full(1, DM),
            full(DM, DP), full(1, DP), full(1, DP), full(1, DP),
        ],
        out_specs=[
            pl.BlockSpec((1, H, K, HD, QB), lambda b, i: (b, 0, 0, 0, i)),
            pl.BlockSpec((1, DP, QB), lambda b, i: (b, 0, i)),
        ],
        out_shape=[
            jax.ShapeDtypeStruct((B, H, K, HD, N), jnp.float32),
            jax.ShapeDtypeStruct((B, DP, N), jnp.float32),
        ],
    )(q, gkv, xyz, feats, *wp)
    return out


def kernel(xyz, features, params):
    p = params
    xyzt = xyz.transpose(0, 2, 1)
    q, tbl, knn = _stage_a(xyz, xyzt, features, p)

    info = plsc.get_sparse_core_info()
    nc, ns = info.num_cores, info.num_subcores
    nw = nc * ns
    tbl2 = tbl.reshape(B * N, 2 * DM)
    idx3 = knn.reshape(nw, (B * N * K) // (nw * 128), 128)
    gkv = _sc_gather(tbl2, idx3, nc, nw)

    z = jnp.zeros((HD, HD), jnp.float32)
    def bd(w):
        rows = []
        for i in range(H):
            rows.append(jnp.concatenate(
                [w if i == j else z for j in range(H)], axis=1))
        return jnp.concatenate(rows, axis=0)
    wp = (
        p['fd1_w'], p['fd1_b'].reshape(1, DM), p['fd2_w'],
        p['fd2_b'].reshape(1, DM),
        bd(p['fg1_w']), jnp.tile(p['fg1_b'], H).reshape(1, DM),
        bd(p['fg2_w']), jnp.tile(p['fg2_b'], H).reshape(1, DM),
        p['proj'], p['n1_g'].reshape(1, DM), p['n1_b'].reshape(1, DM),
        p['fc2_w'], p['fc2_b'].reshape(1, DP),
        p['n2_g'].reshape(1, DP), p['n2_b'].reshape(1, DP),
    )
    attn5, res_t = _stage_b(q, gkv, xyz, features, wp)
    attn = attn5.transpose(0, 1, 4, 2, 3).reshape(B * H, N, K, HD)
    res = res_t.transpose(0, 2, 1)
    return res, attn
